# baseline (speedup 1.0000x reference)
import functools

import jax
import jax.numpy as jnp
from jax import lax
from jax.experimental import pallas as pl
from jax.experimental.pallas import tpu as pltpu
from jax.experimental.pallas import tpu_sc as plsc

NW = 32
NBUF = 8


@functools.lru_cache(maxsize=None)
def _build(batch, seq, D):
    assert batch % (NW * NBUF) == 0
    seqp = (seq + 7) // 8 * 8
    bpw = batch // NW
    ngroup = bpw // NBUF
    mesh = plsc.VectorSubcoreMesh(core_axis_name="c", subcore_axis_name="s")

    @functools.partial(
        pl.kernel,
        mesh=mesh,
        compiler_params=pltpu.CompilerParams(use_tc_tiling_on_sc=False),
        out_type=jax.ShapeDtypeStruct((batch, seq, D), jnp.float32),
        scratch_types=[
            pltpu.VMEM((bpw, seqp), jnp.int32),
            pltpu.VMEM((bpw, seqp), jnp.int32),
            pltpu.VMEM((bpw, seqp), jnp.int32),
            pltpu.VMEM((NBUF, seqp, D), jnp.float32),
            pltpu.SemaphoreType.DMA((NBUF,)),
            pltpu.SemaphoreType.DMA((NBUF,)),
            pltpu.SemaphoreType.DMA((NBUF,)),
        ],
    )
    def embed(eidx_hbm, pidx_hbm, sidx_hbm, etab, ptab, stab, out,
              eidx, pidx, sidx, buf, sem_e, sem_a, sem_o):
        wid = lax.axis_index("s") * 2 + lax.axis_index("c")
        base = wid * bpw
        pltpu.sync_copy(eidx_hbm.at[pl.ds(base, bpw)], eidx)
        pltpu.sync_copy(pidx_hbm.at[pl.ds(base, bpw)], pidx)
        pltpu.sync_copy(sidx_hbm.at[pl.ds(base, bpw)], sidx)

        def group_body(g, carry):
            grow = g * NBUF
            ce = []
            for b in range(NBUF):
                ce.append(pltpu.async_copy(
                    etab.at[eidx.at[grow + b]], buf.at[b], sem_e.at[b]))
            ca = []
            for b in range(NBUF):
                ce[b].wait()
                ca.append(pltpu.async_copy(
                    ptab.at[pidx.at[grow + b]], buf.at[b],
                    sem_a.at[b], add=True))
                ca.append(pltpu.async_copy(
                    stab.at[sidx.at[grow + b]], buf.at[b],
                    sem_a.at[b], add=True))
            co = []
            for b in range(NBUF):
                ca[2 * b].wait()
                ca[2 * b + 1].wait()
                co.append(pltpu.async_copy(
                    buf.at[b, pl.ds(0, seq)], out.at[base + grow + b],
                    sem_o.at[b]))
            for b in range(NBUF):
                co[b].wait()
            return carry

        lax.fori_loop(0, ngroup, group_body, 0)

    return embed


def kernel(inp, pref, suffixes, chrs, input_table, prefix_table, suffix_table):
    batch, seq = inp.shape
    D = input_table.shape[1]
    seqp = (seq + 7) // 8 * 8
    pad = ((0, 0), (0, seqp - seq))
    e = jnp.pad(inp.astype(jnp.int32), pad)
    p = jnp.pad(pref.astype(jnp.int32), pad)
    s = jnp.pad(suffixes.astype(jnp.int32), pad)
    return _build(batch, seq, D)(e, p, s, input_table, prefix_table,
                                 suffix_table)


# speedup vs baseline: 1.8754x; 1.8754x over previous
import functools

import jax
import jax.numpy as jnp
from jax import lax
from jax.experimental import pallas as pl
from jax.experimental.pallas import tpu as pltpu
from jax.experimental.pallas import tpu_sc as plsc

NW = 32
NU = 3


@functools.lru_cache(maxsize=None)
def _build(batch, seq, D):
    assert batch % (NW * 128) == 0 and D % 8 == 0
    bpw = batch // NW
    dblks = D // 8
    npair = seq // 2
    ngroup = npair // NU
    ntail = npair - ngroup * NU
    tiles = seq * dblks * (batch // 128)
    mesh = plsc.VectorSubcoreMesh(core_axis_name="c", subcore_axis_name="s")

    @functools.partial(
        pl.kernel,
        mesh=mesh,
        compiler_params=pltpu.CompilerParams(use_tc_tiling_on_sc=False,
                                             needs_layout_passes=False),
        out_type=jax.ShapeDtypeStruct((tiles, 1024), jnp.float32),
        scratch_types=[
            pltpu.VMEM((bpw, seq), jnp.int32),
            pltpu.VMEM((seq, bpw), jnp.int32),
            pltpu.VMEM((seq, bpw), jnp.int32),
            pltpu.VMEM((seq, bpw), jnp.int32),
            pltpu.VMEM((NU, 2, bpw, D), jnp.float32),
            pltpu.VMEM((NU, 2 * dblks, 1024), jnp.float32),
            pltpu.VMEM((NU, 16), jnp.int32),
            pltpu.SemaphoreType.DMA((NU, 2)),
            pltpu.SemaphoreType.DMA((NU, 2)),
            pltpu.SemaphoreType.DMA((NU,)),
        ],
    )
    def embed(eidx_hbm, pidx_hbm, sidx_hbm, etab, ptab, stab, out,
              raw, eidx, pidx, sidx, buf, bufT, tidb, sem_e, sem_a, sem_o):
        wid = lax.axis_index("s") * 2 + lax.axis_index("c")
        iota = lax.iota(jnp.int32, 16)

        for src_hbm, dst in ((eidx_hbm, eidx), (pidx_hbm, pidx),
                             (sidx_hbm, sidx)):
            pltpu.sync_copy(src_hbm.at[pl.ds(wid * bpw, bpw)], raw)

            def trans_idx(t, c, dst=dst):
                tv = jnp.zeros((16,), jnp.int32) + t
                for k in range(bpw // 16):
                    v = plsc.load_gather(raw, [iota + 16 * k, tv])
                    dst[t, pl.ds(16 * k, 16)] = v
                return c

            lax.fori_loop(0, seq, trans_idx, 0)

        def do_pair(u, t0):
            def gathers_e():
                return [pltpu.async_copy(
                    etab.at[eidx.at[t0 + cc]], buf.at[u, cc],
                    sem_e.at[u, cc]) for cc in range(2)]

            def gathers_ps(ce):
                ca = []
                for cc in range(2):
                    ce[cc].wait()
                    ca.append(pltpu.async_copy(
                        ptab.at[pidx.at[t0 + cc]], buf.at[u, cc],
                        sem_a.at[u, cc], add=True))
                    ca.append(pltpu.async_copy(
                        stab.at[sidx.at[t0 + cc]], buf.at[u, cc],
                        sem_a.at[u, cc], add=True))
                return ca

            def finish(ca):
                for cc in range(2):
                    ca[2 * cc].wait()
                    ca[2 * cc + 1].wait()
                for cc in range(2):
                    def trans(d, c, cc=cc):
                        dv = jnp.zeros((16,), jnp.int32) + d
                        row = cc * dblks + d // 8
                        col = (d % 8) * 128
                        for k in range(bpw // 16):
                            v = plsc.load_gather(
                                buf.at[u, cc], [iota + 16 * k, dv])
                            bufT[u, row, pl.ds(col + 16 * k, 16)] = v
                        return c

                    lax.fori_loop(0, D, trans, 0)
                tidb[u, pl.ds(0, 16)] = iota * 32 + (t0 * (dblks * 32) + wid)
                return pltpu.async_copy(
                    bufT.at[u], out.at[tidb.at[u]], sem_o.at[u])

            return gathers_e, gathers_ps, finish

        def group_body(g, carry):
            stages = [do_pair(u, (g * NU + u) * 2) for u in range(NU)]
            ce = [st[0]() for st in stages]
            ca = [st[1](ce[u]) for u, st in enumerate(stages)]
            cs = [st[2](ca[u]) for u, st in enumerate(stages)]
            for c in cs:
                c.wait()
            return carry

        lax.fori_loop(0, ngroup, group_body, 0)
        if ntail:
            for u in range(ntail):
                stages = do_pair(u, (ngroup * NU + u) * 2)
                ce = stages[0]()
                ca = stages[1](ce)
                cs = stages[2](ca)
                cs.wait()

    return embed


def kernel(inp, pref, suffixes, chrs, input_table, prefix_table, suffix_table):
    batch, seq = inp.shape
    D = input_table.shape[1]
    e = inp.astype(jnp.int32)
    p = pref.astype(jnp.int32)
    s = suffixes.astype(jnp.int32)
    phys = _build(batch, seq, D)(e, p, s, input_table, prefix_table,
                                 suffix_table)
    return (phys.reshape(seq, D // 8, batch // 128, 8, 128)
            .transpose(2, 4, 0, 1, 3)
            .reshape(batch, seq, D))


# speedup vs baseline: 2.3668x; 1.2620x over previous
import functools

import jax
import jax.numpy as jnp
from jax import lax
from jax.experimental import pallas as pl
from jax.experimental.pallas import tpu as pltpu
from jax.experimental.pallas import tpu_sc as plsc

NW = 32
NU = 2
PITCH = 65


@functools.lru_cache(maxsize=None)
def _build(batch, seq, D):
    assert batch % (NW * 128) == 0 and D % 8 == 0
    bpw = batch // NW
    dblks = D // 8
    npair = seq // 2
    ngroup = npair // NU
    ntail = npair - ngroup * NU
    tiles = seq * dblks * (batch // 128)
    mesh = plsc.VectorSubcoreMesh(core_axis_name="c", subcore_axis_name="s")

    @functools.partial(
        pl.kernel,
        mesh=mesh,
        compiler_params=pltpu.CompilerParams(use_tc_tiling_on_sc=False,
                                             needs_layout_passes=False),
        out_type=jax.ShapeDtypeStruct((tiles, 1024), jnp.float32),
        scratch_types=[
            pltpu.VMEM((bpw, seq), jnp.int32),
            pltpu.VMEM((seq, bpw), jnp.int32),
            pltpu.VMEM((seq, bpw), jnp.int32),
            pltpu.VMEM((seq, bpw), jnp.int32),
            pltpu.VMEM((NU, 2, bpw, D), jnp.float32),
            pltpu.VMEM((NU, 2 * dblks, 1024), jnp.float32),
            pltpu.VMEM((128 * PITCH,), jnp.float32),
            pltpu.VMEM((NU, 16), jnp.int32),
            pltpu.SemaphoreType.DMA((NU, 2)),
            pltpu.SemaphoreType.DMA((NU, 2)),
            pltpu.SemaphoreType.DMA((NU,)),
        ],
    )
    def embed(eidx_hbm, pidx_hbm, sidx_hbm, etab, ptab, stab, out,
              raw, eidx, pidx, sidx, buf, bufT, stg, tidb, sem_e, sem_a, sem_o):
        wid = lax.axis_index("s") * 2 + lax.axis_index("c")
        iota = lax.iota(jnp.int32, 16)
        iotap = iota * PITCH

        for src_hbm, dst in ((eidx_hbm, eidx), (pidx_hbm, pidx),
                             (sidx_hbm, sidx)):
            pltpu.sync_copy(src_hbm.at[pl.ds(wid * bpw, bpw)], raw)

            def trans_idx(t, c, dst=dst):
                tv = jnp.zeros((16,), jnp.int32) + t
                for k in range(bpw // 16):
                    v = plsc.load_gather(raw, [iota + 16 * k, tv])
                    dst[t, pl.ds(16 * k, 16)] = v
                return c

            lax.fori_loop(0, seq, trans_idx, 0)

        def do_pair(u, t0):
            def gathers_e():
                return [pltpu.async_copy(
                    etab.at[eidx.at[t0 + cc]], buf.at[u, cc],
                    sem_e.at[u, cc]) for cc in range(2)]

            def gathers_ps(ce):
                ca = []
                for cc in range(2):
                    ce[cc].wait()
                    ca.append(pltpu.async_copy(
                        ptab.at[pidx.at[t0 + cc]], buf.at[u, cc],
                        sem_a.at[u, cc], add=True))
                    ca.append(pltpu.async_copy(
                        stab.at[sidx.at[t0 + cc]], buf.at[u, cc],
                        sem_a.at[u, cc], add=True))
                return ca

            def finish(ca):
                for cc in range(2):
                    ca[2 * cc].wait()
                    ca[2 * cc + 1].wait()
                for cc in range(2):
                    def stage(j, c, cc=cc):
                        for k in range(D // 16):
                            v = buf[u, cc, j, pl.ds(16 * k, 16)]
                            plsc.store_scatter(
                                stg, [iota + (j * PITCH + 16 * k)], v)
                        return c

                    lax.fori_loop(0, bpw, stage, 0)

                    def trans(d, c, cc=cc):
                        row = cc * dblks + d // 8
                        col = (d % 8) * 128
                        for k in range(bpw // 16):
                            v = plsc.load_gather(
                                stg, [iotap + (16 * k * PITCH + d)])
                            bufT[u, row, pl.ds(col + 16 * k, 16)] = v
                        return c

                    lax.fori_loop(0, D, trans, 0)
                tidb[u, pl.ds(0, 16)] = iota * 32 + (t0 * (dblks * 32) + wid)
                return pltpu.async_copy(
                    bufT.at[u], out.at[tidb.at[u]], sem_o.at[u])

            return gathers_e, gathers_ps, finish

        def group_body(g, carry):
            stages = [do_pair(u, (g * NU + u) * 2) for u in range(NU)]
            ce = [st[0]() for st in stages]
            ca = [st[1](ce[u]) for u, st in enumerate(stages)]
            cs = [st[2](ca[u]) for u, st in enumerate(stages)]
            for c in cs:
                c.wait()
            return carry

        lax.fori_loop(0, ngroup, group_body, 0)
        if ntail:
            for u in range(ntail):
                stages = do_pair(u, (ngroup * NU + u) * 2)
                ce = stages[0]()
                ca = stages[1](ce)
                cs = stages[2](ca)
                cs.wait()

    return embed


def kernel(inp, pref, suffixes, chrs, input_table, prefix_table, suffix_table):
    batch, seq = inp.shape
    D = input_table.shape[1]
    e = inp.astype(jnp.int32)
    p = pref.astype(jnp.int32)
    s = suffixes.astype(jnp.int32)
    phys = _build(batch, seq, D)(e, p, s, input_table, prefix_table,
                                 suffix_table)
    return (phys.reshape(seq, D // 8, batch // 128, 8, 128)
            .transpose(2, 4, 0, 1, 3)
            .reshape(batch, seq, D))


# speedup vs baseline: 2.9447x; 1.2442x over previous
import functools

import jax
import jax.numpy as jnp
from jax import lax
from jax.experimental import pallas as pl
from jax.experimental.pallas import tpu as pltpu
from jax.experimental.pallas import tpu_sc as plsc

NW = 32
CHUNK = 128
NBUF = 10


@functools.lru_cache(maxsize=None)
def _build(B, D):
    assert B % (NW * CHUNK * NBUF) == 0
    bpw = B // NW
    ngroup = bpw // (CHUNK * NBUF)
    mesh = plsc.VectorSubcoreMesh(core_axis_name="c", subcore_axis_name="s")

    @functools.partial(
        pl.kernel,
        mesh=mesh,
        compiler_params=pltpu.CompilerParams(use_tc_tiling_on_sc=False),
        out_type=jax.ShapeDtypeStruct((B, D), jnp.float32),
        scratch_types=[
            pltpu.VMEM((bpw,), jnp.int32),
            pltpu.VMEM((bpw,), jnp.int32),
            pltpu.VMEM((bpw,), jnp.int32),
            pltpu.VMEM((NBUF, CHUNK, D), jnp.float32),
            pltpu.SemaphoreType.DMA((NBUF,)),
            pltpu.SemaphoreType.DMA((NBUF,)),
            pltpu.SemaphoreType.DMA((NBUF,)),
        ],
    )
    def embed(eidx_hbm, pidx_hbm, sidx_hbm, etab, ptab, stab, out,
              eidx, pidx, sidx, buf, sem_e, sem_a, sem_o):
        wid = lax.axis_index("s") * 2 + lax.axis_index("c")
        base = wid * bpw
        pltpu.sync_copy(eidx_hbm.at[pl.ds(base, bpw)], eidx)
        pltpu.sync_copy(pidx_hbm.at[pl.ds(base, bpw)], pidx)
        pltpu.sync_copy(sidx_hbm.at[pl.ds(base, bpw)], sidx)

        def group_body(g, carry):
            goff = g * (NBUF * CHUNK)
            ce = []
            for b in range(NBUF):
                off = goff + b * CHUNK
                ce.append(pltpu.async_copy(
                    etab.at[eidx.at[pl.ds(off, CHUNK)]], buf.at[b],
                    sem_e.at[b]))
            ca = []
            for b in range(NBUF):
                off = goff + b * CHUNK
                ce[b].wait()
                ca.append(pltpu.async_copy(
                    ptab.at[pidx.at[pl.ds(off, CHUNK)]], buf.at[b],
                    sem_a.at[b], add=True))
                ca.append(pltpu.async_copy(
                    stab.at[sidx.at[pl.ds(off, CHUNK)]], buf.at[b],
                    sem_a.at[b], add=True))
            co = []
            for b in range(NBUF):
                off = goff + b * CHUNK
                ca[2 * b].wait()
                ca[2 * b + 1].wait()
                co.append(pltpu.async_copy(
                    buf.at[b], out.at[pl.ds(base + off, CHUNK)], sem_o.at[b]))
            for b in range(NBUF):
                co[b].wait()
            return carry

        lax.fori_loop(0, ngroup, group_body, 0)

    return embed


def kernel(inp, pref, suffixes, chrs, input_table, prefix_table, suffix_table):
    batch, seq = inp.shape
    D = input_table.shape[1]
    B = batch * seq
    e = inp.reshape(B).astype(jnp.int32)
    p = pref.reshape(B).astype(jnp.int32)
    s = suffixes.reshape(B).astype(jnp.int32)
    out = _build(B, D)(e, p, s, input_table, prefix_table, suffix_table)
    return out.reshape(batch, seq, D)
